# baseline (speedup 1.0000x reference)
import functools

import jax
import jax.numpy as jnp
import numpy as np
from jax.experimental import pallas as pl
from jax.experimental.pallas import tpu as pltpu


def _fused_kernel(x_ref, wc_ref, ssel_ref, wb2_ref, u2_ref, bias_ref, ba_ref,
                  m_ref, am_ref, o_ref, *, W, Wa, HW, Ma, B, Cout):
    P = B // 2
    parts_all = []
    for p in range(P):
        xv = x_ref[p].astype(jnp.float32)
        parts = []
        for t in range(9):
            kh, kw = divmod(t, 3)
            off = (kh - 1) * W + (kw - 1)
            r = pltpu.roll(xv, (-off) % HW, axis=1) if off != 0 else xv
            if t != 4:
                r = r * m_ref[t]
            parts.append(r)
        parts_all.append(jnp.concatenate(parts, axis=0))
    p_all = jnp.concatenate(parts_all, axis=1)

    acc9 = jnp.dot(wc_ref[...], p_all, preferred_element_type=jnp.float32)

    v_all = jnp.concatenate(
        [acc9[16 * (b % 2) + Cout:16 * (b % 2) + Cout + 1,
              (b // 2) * HW:(b // 2 + 1) * HW] for b in range(B)],
        axis=0)
    a_all = jnp.dot(v_all, ssel_ref[...],
                    preferred_element_type=jnp.float32) + ba_ref[...]

    aparts = []
    for t in range(9):
        kh, kw = divmod(t, 3)
        off = (kh - 1) * Wa + (kw - 1)
        r = pltpu.roll(a_all, (-off) % Ma, axis=1) if off != 0 else a_all
        if t != 4:
            r = r * am_ref[t]
        aparts.append(r)
    a9 = jnp.concatenate(aparts, axis=0)

    ls = jnp.dot(wb2_ref[...], a9, preferred_element_type=jnp.float32)
    up = jnp.dot(ls.astype(jnp.bfloat16), u2_ref[...],
                 preferred_element_type=jnp.float32)

    for b in range(B):
        o_ref[b] = (acc9[16 * (b % 2):16 * (b % 2) + Cout,
                         (b // 2) * HW:(b // 2 + 1) * HW]
                    + up[b * Cout:(b + 1) * Cout]
                    + bias_ref[...]).astype(o_ref.dtype)


def kernel(x, w_fixed, b_fixed, w_a, b_a, w_b, b_b):
    N, Cin, H, W = x.shape
    Cout = w_fixed.shape[0]
    HW = H * W
    Ha, Wa = H // 4, W // 4
    Ma = Ha * Wa
    dtype = x.dtype
    bf = jnp.bfloat16

    Nin = N
    if N % 2:
        x = jnp.pad(x, ((0, 1), (0, 0), (0, 0), (0, 0)))
        N = N + 1
    xf = x.astype(bf).reshape(N // 2, 2 * Cin, HW)

    wc3 = jnp.concatenate([
        jnp.transpose(w_fixed, (0, 2, 3, 1)).reshape(Cout, 9, Cin),
        jnp.transpose(w_a, (0, 2, 3, 1)).reshape(1, 9, Cin),
    ], axis=0)
    wc = jnp.zeros((2, 16, 9, 2, Cin), jnp.float32)
    wc = wc.at[0, :Cout + 1, :, 0, :].set(wc3)
    wc = wc.at[1, :Cout + 1, :, 1, :].set(wc3)
    wc = wc.reshape(32, 9 * 2 * Cin)
    wb9 = w_b.reshape(Cout, 9)
    bias = (b_fixed + b_b).reshape(Cout, 1)
    ba = b_a.reshape(1, 1)

    hh = np.arange(HW) // W
    ww = np.arange(HW) % W
    ha = np.arange(Ma) // Wa
    wa_ = np.arange(Ma) % Wa
    masks, amasks = [], []
    for t in range(9):
        kh, kw = divmod(t, 3)
        masks.append(((hh + kh - 1 >= 0) & (hh + kh - 1 < H)
                      & (ww + kw - 1 >= 0) & (ww + kw - 1 < W)))
        amasks.append(((ha + kh - 1 >= 0) & (ha + kh - 1 < Ha)
                       & (wa_ + kw - 1 >= 0) & (wa_ + kw - 1 < Wa)))
    m9 = jnp.asarray(np.stack(masks).reshape(9, 1, HW).astype(np.float32))
    am9 = jnp.asarray(np.stack(amasks).reshape(9, 1, Ma).astype(np.float32))

    q_of_m = (ha * 4) * W + wa_ * 4
    ssel = jnp.asarray(
        (np.arange(HW)[:, None] == q_of_m[None, :]).astype(np.float32))
    m_of_q = (hh // 4) * Wa + ww // 4
    u2 = jnp.asarray(
        (np.arange(Ma)[:, None] == m_of_q[None, :]).astype(np.float32),
        dtype=jnp.bfloat16)

    flops = int(N * (2 * (Cout + 1) * Cin * 9 * HW + 2 * HW * Ma
                     + 2 * Cout * 9 * Ma + 2 * Cout * Ma * HW))
    bytes_accessed = int(4 * (N * Cin * HW + N * Cout * HW)
                         + 2 * (HW * Ma * 2 + 9 * HW + 9 * Ma))

    B = 16
    while N % B:
        B //= 2

    eyeb = jnp.eye(B, dtype=jnp.float32)
    wb2 = (wb9[None, :, :, None] * eyeb[:, None, None, :]).reshape(
        B * Cout, 9 * B)

    kern = functools.partial(_fused_kernel, W=W, Wa=Wa, HW=HW, Ma=Ma, B=B,
                             Cout=Cout)
    out = pl.pallas_call(
        kern,
        out_shape=jax.ShapeDtypeStruct((N, Cout, HW), dtype),
        grid=(N // B,),
        in_specs=[
            pl.BlockSpec((B // 2, 2 * Cin, HW), lambda n: (n, 0, 0)),
            pl.BlockSpec((32, 9 * 2 * Cin), lambda n: (0, 0)),
            pl.BlockSpec((HW, Ma), lambda n: (0, 0)),
            pl.BlockSpec((B * Cout, 9 * B), lambda n: (0, 0)),
            pl.BlockSpec((Ma, HW), lambda n: (0, 0)),
            pl.BlockSpec((Cout, 1), lambda n: (0, 0)),
            pl.BlockSpec((1, 1), lambda n: (0, 0)),
            pl.BlockSpec((9, 1, HW), lambda n: (0, 0, 0)),
            pl.BlockSpec((9, 1, Ma), lambda n: (0, 0, 0)),
        ],
        out_specs=pl.BlockSpec((B, Cout, HW), lambda n: (n, 0, 0)),
        compiler_params=pltpu.CompilerParams(dimension_semantics=("parallel",)),
        cost_estimate=pl.CostEstimate(flops=flops, transcendentals=0,
                                      bytes_accessed=bytes_accessed),
    )(xf, wc, ssel, wb2, u2, bias, ba, m9, am9)

    return out.reshape(N, Cout, H, W)[:Nin]


# speedup vs baseline: 1.0597x; 1.0597x over previous
import functools

import jax
import jax.numpy as jnp
import numpy as np
from jax.experimental import pallas as pl
from jax.experimental.pallas import tpu as pltpu


def _fused_kernel(x_ref, wc_ref, ssel_ref, wb2_ref, u2_ref, bias_ref, ba_ref,
                  m_ref, am_ref, o_ref, *, W, Wa, HW, Ma, B, Cout):
    P = B // 2
    parts_all = []
    for p in range(P):
        xv = x_ref[p].astype(jnp.float32)
        parts = []
        for t in range(9):
            kh, kw = divmod(t, 3)
            off = (kh - 1) * W + (kw - 1)
            r = pltpu.roll(xv, (-off) % HW, axis=1) if off != 0 else xv
            if t != 4:
                r = r * m_ref[t]
            parts.append(r)
        parts_all.append(jnp.concatenate(parts, axis=0))
    p_all = jnp.concatenate(parts_all, axis=1)

    acc9 = jnp.dot(wc_ref[...], p_all, preferred_element_type=jnp.float32)

    v_all = jnp.concatenate(
        [acc9[16 * (b % 2) + Cout:16 * (b % 2) + Cout + 1,
              (b // 2) * HW:(b // 2 + 1) * HW] for b in range(B)],
        axis=0)
    a_all = jnp.dot(v_all, ssel_ref[...],
                    preferred_element_type=jnp.float32) + ba_ref[...]

    aparts = []
    for t in range(9):
        kh, kw = divmod(t, 3)
        off = (kh - 1) * Wa + (kw - 1)
        r = pltpu.roll(a_all, (-off) % Ma, axis=1) if off != 0 else a_all
        if t != 4:
            r = r * am_ref[t]
        aparts.append(r)
    a9 = jnp.concatenate(aparts, axis=0)

    ls = jnp.dot(wb2_ref[...], a9, preferred_element_type=jnp.float32)
    up = jnp.dot(ls, u2_ref[...],
                 preferred_element_type=jnp.float32)

    for b in range(B):
        o_ref[b] = (acc9[16 * (b % 2):16 * (b % 2) + Cout,
                         (b // 2) * HW:(b // 2 + 1) * HW]
                    + up[b * Cout:(b + 1) * Cout]
                    + bias_ref[...]).astype(o_ref.dtype)


def kernel(x, w_fixed, b_fixed, w_a, b_a, w_b, b_b):
    N, Cin, H, W = x.shape
    Cout = w_fixed.shape[0]
    HW = H * W
    Ha, Wa = H // 4, W // 4
    Ma = Ha * Wa
    dtype = x.dtype
    bf = jnp.bfloat16

    Nin = N
    if N % 2:
        x = jnp.pad(x, ((0, 1), (0, 0), (0, 0), (0, 0)))
        N = N + 1
    xf = x.astype(bf).reshape(N // 2, 2 * Cin, HW)

    wc3 = jnp.concatenate([
        jnp.transpose(w_fixed, (0, 2, 3, 1)).reshape(Cout, 9, Cin),
        jnp.transpose(w_a, (0, 2, 3, 1)).reshape(1, 9, Cin),
    ], axis=0)
    wc = jnp.zeros((2, 16, 9, 2, Cin), jnp.float32)
    wc = wc.at[0, :Cout + 1, :, 0, :].set(wc3)
    wc = wc.at[1, :Cout + 1, :, 1, :].set(wc3)
    wc = wc.reshape(32, 9 * 2 * Cin)
    wb9 = w_b.reshape(Cout, 9)
    bias = (b_fixed + b_b).reshape(Cout, 1)
    ba = b_a.reshape(1, 1)

    hh = np.arange(HW) // W
    ww = np.arange(HW) % W
    ha = np.arange(Ma) // Wa
    wa_ = np.arange(Ma) % Wa
    masks, amasks = [], []
    for t in range(9):
        kh, kw = divmod(t, 3)
        masks.append(((hh + kh - 1 >= 0) & (hh + kh - 1 < H)
                      & (ww + kw - 1 >= 0) & (ww + kw - 1 < W)))
        amasks.append(((ha + kh - 1 >= 0) & (ha + kh - 1 < Ha)
                       & (wa_ + kw - 1 >= 0) & (wa_ + kw - 1 < Wa)))
    m9 = jnp.asarray(np.stack(masks).reshape(9, 1, HW).astype(np.float32))
    am9 = jnp.asarray(np.stack(amasks).reshape(9, 1, Ma).astype(np.float32))

    q_of_m = (ha * 4) * W + wa_ * 4
    ssel = jnp.asarray(
        (np.arange(HW)[:, None] == q_of_m[None, :]).astype(np.float32))
    m_of_q = (hh // 4) * Wa + ww // 4
    u2 = jnp.asarray(
        (np.arange(Ma)[:, None] == m_of_q[None, :]).astype(np.float32))

    flops = int(N * (2 * (Cout + 1) * Cin * 9 * HW + 2 * HW * Ma
                     + 2 * Cout * 9 * Ma + 2 * Cout * Ma * HW))
    bytes_accessed = int(4 * (N * Cin * HW + N * Cout * HW)
                         + 2 * (HW * Ma * 2 + 9 * HW + 9 * Ma))

    B = 16
    while N % B:
        B //= 2

    eyeb = jnp.eye(B, dtype=jnp.float32)
    wb2 = (wb9[None, :, :, None] * eyeb[:, None, None, :]).reshape(
        B * Cout, 9 * B)

    kern = functools.partial(_fused_kernel, W=W, Wa=Wa, HW=HW, Ma=Ma, B=B,
                             Cout=Cout)
    out = pl.pallas_call(
        kern,
        out_shape=jax.ShapeDtypeStruct((N, Cout, HW), dtype),
        grid=(N // B,),
        in_specs=[
            pl.BlockSpec((B // 2, 2 * Cin, HW), lambda n: (n, 0, 0)),
            pl.BlockSpec((32, 9 * 2 * Cin), lambda n: (0, 0)),
            pl.BlockSpec((HW, Ma), lambda n: (0, 0)),
            pl.BlockSpec((B * Cout, 9 * B), lambda n: (0, 0)),
            pl.BlockSpec((Ma, HW), lambda n: (0, 0)),
            pl.BlockSpec((Cout, 1), lambda n: (0, 0)),
            pl.BlockSpec((1, 1), lambda n: (0, 0)),
            pl.BlockSpec((9, 1, HW), lambda n: (0, 0, 0)),
            pl.BlockSpec((9, 1, Ma), lambda n: (0, 0, 0)),
        ],
        out_specs=pl.BlockSpec((B, Cout, HW), lambda n: (n, 0, 0)),
        compiler_params=pltpu.CompilerParams(dimension_semantics=("parallel",)),
        cost_estimate=pl.CostEstimate(flops=flops, transcendentals=0,
                                      bytes_accessed=bytes_accessed),
    )(xf, wc, ssel, wb2, u2, bias, ba, m9, am9)

    return out.reshape(N, Cout, H, W)[:Nin]
